# Initial kernel scaffold; baseline (speedup 1.0000x reference)
import jax
import jax.numpy as jnp
from jax.experimental import pallas as pl


def kernel(x, queue_buf):
    raise NotImplementedError("write your pallas kernel here")


# speedup vs baseline: 22.5931x; 22.5931x over previous
import functools

import jax
import jax.numpy as jnp
from jax import lax
from jax.experimental import pallas as pl
from jax.experimental.pallas import tpu as pltpu
from jax.experimental.pallas import tpu_sc as plsc

N = 4096
D = 16
BQ = 512
GRID = N // BQ


def _nn_idx_body(q_ref, x_ref, idx_ref):
    i = pl.program_id(0)
    q = q_ref[...]
    xf = x_ref[...]
    g = lax.dot_general(q, xf, (((1,), (1,)), ((), ())),
                        preferred_element_type=jnp.float32)
    q2 = jnp.sum(q * q, axis=1, keepdims=True)
    x2 = jnp.sum(xf * xf, axis=1)
    d2 = q2 + x2[None, :] - 2.0 * g
    d2 = jnp.maximum(d2, 0.0)
    cols = lax.broadcasted_iota(jnp.int32, (BQ, N), 1)
    rows = i * BQ + lax.broadcasted_iota(jnp.int32, (BQ, N), 0)
    d2 = jnp.where(cols == rows, jnp.inf, d2)
    m = jnp.min(d2, axis=1, keepdims=True)
    cand = jnp.where(d2 == m, cols, N)
    idx = jnp.min(cand, axis=1).astype(jnp.int32)
    idx_ref[...] = idx.reshape(1, 1, BQ)


def _nn_indices(x):
    idx3 = pl.pallas_call(
        _nn_idx_body,
        grid=(GRID,),
        in_specs=[
            pl.BlockSpec((BQ, D), lambda i: (i, 0)),
            pl.BlockSpec((N, D), lambda i: (0, 0)),
        ],
        out_specs=pl.BlockSpec((1, 1, BQ), lambda i: (i, 0, 0)),
        out_shape=jax.ShapeDtypeStruct((GRID, 1, BQ), jnp.int32),
    )(x, x)
    return idx3.reshape(N)


def _make_sc_gather():
    info = plsc.get_sparse_core_info()
    nw = info.num_cores * info.num_subcores
    b_per_w = N // nw
    mesh = plsc.VectorSubcoreMesh(core_axis_name="c", subcore_axis_name="s")

    @functools.partial(
        pl.kernel,
        mesh=mesh,
        compiler_params=pltpu.CompilerParams(use_tc_tiling_on_sc=False),
        out_type=jax.ShapeDtypeStruct((N, D), jnp.float32),
        scratch_types=[
            pltpu.VMEM((b_per_w,), jnp.int32),
            pltpu.VMEM((b_per_w, D), jnp.float32),
            pltpu.SemaphoreType.DMA,
        ],
    )
    def gather(table_hbm, idx_hbm, out_hbm, idx_v, rows_v, sem):
        wid = lax.axis_index("s") * info.num_cores + lax.axis_index("c")
        base = wid * b_per_w
        pltpu.sync_copy(idx_hbm.at[pl.ds(base, b_per_w)], idx_v)
        pltpu.async_copy(table_hbm.at[idx_v], rows_v, sem).wait()
        pltpu.sync_copy(rows_v, out_hbm.at[pl.ds(base, b_per_w)])

    return gather


_sc_gather = None


def kernel(x, queue_buf):
    del queue_buf
    global _sc_gather
    if _sc_gather is None:
        _sc_gather = _make_sc_gather()
    idx = _nn_indices(x)
    return _sc_gather(x, idx)


# speedup vs baseline: 22.8052x; 1.0094x over previous
import functools

import jax
import jax.numpy as jnp
from jax import lax
from jax.experimental import pallas as pl
from jax.experimental.pallas import tpu as pltpu
from jax.experimental.pallas import tpu_sc as plsc

N = 4096
D = 16
BQ = 1024
GRID = N // BQ
INF = float("inf")


def _nn_idx_body(q_ref, x_ref, idx_ref, x2_ref):
    i = pl.program_id(0)

    @pl.when(i == 0)
    def _build_x2():
        xf = x_ref[...]
        x2 = jnp.sum(xf * xf, axis=1, keepdims=True)
        x2_ref[...] = x2.reshape(1, N)

    q = q_ref[...]
    g = lax.dot_general(q, x_ref[...], (((1,), (1,)), ((), ())),
                        preferred_element_type=jnp.float32)
    q2 = jnp.sum(q * q, axis=1, keepdims=True)
    x2l = x2_ref[...]
    d2 = jnp.maximum(q2 + x2l - 2.0 * g, 0.0)
    cols = lax.broadcasted_iota(jnp.int32, (BQ, N), 1)
    rows = i * BQ + lax.broadcasted_iota(jnp.int32, (BQ, N), 0)
    d2 = jnp.where(cols == rows, INF, d2)
    m = jnp.min(d2, axis=1, keepdims=True)
    idx = jnp.min(jnp.where(d2 <= m, cols, N), axis=1).astype(jnp.int32)
    idx_ref[...] = idx.reshape(1, 1, BQ)


def _nn_indices(x):
    return pl.pallas_call(
        _nn_idx_body,
        grid=(GRID,),
        in_specs=[
            pl.BlockSpec((BQ, D), lambda i: (i, 0)),
            pl.BlockSpec((N, D), lambda i: (0, 0)),
        ],
        out_specs=pl.BlockSpec((1, 1, BQ), lambda i: (i, 0, 0)),
        out_shape=jax.ShapeDtypeStruct((GRID, 1, BQ), jnp.int32),
        scratch_shapes=[pltpu.VMEM((1, N), jnp.float32)],
    )(x, x)


def _make_sc_gather():
    info = plsc.get_sparse_core_info()
    nw = info.num_cores * info.num_subcores
    b_per_w = N // nw
    wpg = BQ // b_per_w
    mesh = plsc.VectorSubcoreMesh(core_axis_name="c", subcore_axis_name="s")

    @functools.partial(
        pl.kernel,
        mesh=mesh,
        compiler_params=pltpu.CompilerParams(use_tc_tiling_on_sc=False),
        out_type=jax.ShapeDtypeStruct((N, D), jnp.float32),
        scratch_types=[
            pltpu.VMEM((b_per_w,), jnp.int32),
            pltpu.VMEM((b_per_w, D), jnp.float32),
            pltpu.SemaphoreType.DMA,
        ],
    )
    def gather(table_hbm, idx_hbm, out_hbm, idx_v, rows_v, sem):
        wid = lax.axis_index("s") * info.num_cores + lax.axis_index("c")
        g = wid // wpg
        off = (wid % wpg) * b_per_w
        pltpu.sync_copy(idx_hbm.at[g, 0, pl.ds(off, b_per_w)], idx_v)
        pltpu.async_copy(table_hbm.at[idx_v], rows_v, sem).wait()
        pltpu.sync_copy(rows_v, out_hbm.at[pl.ds(wid * b_per_w, b_per_w)])

    return gather


_sc_gather = None


def kernel(x, queue_buf):
    del queue_buf
    global _sc_gather
    if _sc_gather is None:
        _sc_gather = _make_sc_gather()
    idx3 = _nn_indices(x)
    return _sc_gather(x, idx3)
